# baseline (speedup 1.0000x reference)
import functools

import jax
import jax.numpy as jnp
from jax import lax
from jax.experimental import pallas as pl
from jax.experimental.pallas import tpu as pltpu
from jax.experimental.pallas import tpu_sc as plsc

NC = 2
NS = 16
NW = NC * NS
L = 16


def _sc_mesh():
    return plsc.VectorSubcoreMesh(
        core_axis_name="c", subcore_axis_name="s", num_cores=NC,
        num_subcores=NS)


@functools.lru_cache(maxsize=None)
def _make_scale_kernel(n_edges: int, n_seg: int):
    EPW = n_edges // NW
    CPW = n_edges // NS
    assert EPW % L == 0 and n_seg % L == 0
    RW = 128
    NROW = pl.cdiv(n_seg, RW)
    NROW_PAD = pl.cdiv(NROW, RW) * RW
    SEG_PAD = NROW_PAD * RW
    RPT = NROW_PAD // NS

    @functools.partial(
        pl.kernel,
        out_type=jax.ShapeDtypeStruct((n_edges,), jnp.float32),
        mesh=_sc_mesh(),
        scratch_types=[
            pltpu.VMEM_SHARED((NROW_PAD, RW), jnp.float32),
            pltpu.VMEM((NROW_PAD, RW), jnp.float32),
            pltpu.VMEM((CPW,), jnp.int32),
            pltpu.VMEM((EPW,), jnp.float32),
            pltpu.VMEM((RW,), jnp.int32),
            pltpu.VMEM((RPT, RW), jnp.float32),
        ],
        compiler_params=pltpu.CompilerParams(needs_layout_passes=False),
    )
    def scale_kernel(cidx_hbm, zeros_hbm, scale_hbm, shared_sh, counts_v,
                     stage_v, sbuf_v, ridx_v, work_v):
        cc = lax.axis_index("c")
        ss = lax.axis_index("s")
        wid = cc * NS + ss
        base = wid * EPW

        pltpu.sync_copy(zeros_hbm, counts_v)
        pltpu.sync_copy(zeros_hbm.at[pl.ds(ss * RPT, RPT)], work_v)
        pltpu.sync_copy(work_v, shared_sh.at[pl.ds(ss * RPT, RPT)])

        pltpu.sync_copy(cidx_hbm.at[pl.ds(ss * CPW, CPW)], stage_v)
        ones = jnp.ones((L,), jnp.float32)
        UNR = 5
        assert CPW % (L * UNR) == 0

        def count_vec(j, c2):
            for u in range(UNR):
                idx = stage_v[pl.ds((j * UNR + u) * L, L)]
                plsc.addupdate_scatter(
                    counts_v, [lax.shift_right_logical(idx, 7),
                               lax.bitwise_and(idx, RW - 1)], ones)
            return c2

        lax.fori_loop(0, CPW // (L * UNR), count_vec, 0)
        plsc.subcore_barrier()

        def reduce_chunk(r, carry):
            rbase = r * RW
            for v in range(RW // L):
                ridx_v[pl.ds(v * L, L)] = (
                    lax.iota(jnp.int32, L) + rbase + v * L)
            pltpu.sync_copy(counts_v.at[pl.ds(rbase, RW)],
                            shared_sh.at[ridx_v], add=True)
            return carry

        lax.fori_loop(0, NROW_PAD // RW, reduce_chunk, 0)
        plsc.subcore_barrier()

        pltpu.sync_copy(shared_sh.at[pl.ds(ss * RPT, RPT)], work_v)

        def inv_row(r, carry):
            for v in range(RW // L):
                val = work_v[r, pl.ds(v * L, L)]
                work_v[r, pl.ds(v * L, L)] = 1.0 / jnp.maximum(val, 1.0)
            return carry

        lax.fori_loop(0, RPT, inv_row, 0)
        pltpu.sync_copy(work_v, shared_sh.at[pl.ds(ss * RPT, RPT)])
        plsc.subcore_barrier()

        pltpu.sync_copy(shared_sh, counts_v)
        pltpu.sync_copy(cidx_hbm.at[pl.ds(base, EPW)],
                        stage_v.at[pl.ds(0, EPW)])

        def scale_vec(i, carry):
            idx = stage_v[pl.ds(i * L, L)]
            sbuf_v[pl.ds(i * L, L)] = plsc.load_gather(
                counts_v, [lax.shift_right_logical(idx, 7),
                           lax.bitwise_and(idx, RW - 1)])
            return carry

        lax.fori_loop(0, EPW // L, scale_vec, 0)
        pltpu.sync_copy(sbuf_v, scale_hbm.at[pl.ds(base, EPW)])

    return scale_kernel


@functools.lru_cache(maxsize=None)
def _make_agg_kernel(tbl_rows: int, n_nodes: int, hidden: int,
                     n_edges: int):
    K = 80
    EPW = n_edges // NW
    assert EPW % K == 0 and K % 8 == 0
    NCHUNK = EPW // K
    ZR = 80
    NZCH = pl.cdiv(n_nodes, ZR)
    NZROUND = pl.cdiv(NZCH, NS)
    assert n_nodes % ZR == 0
    HV = hidden // L

    S = 2000
    assert EPW % S == 0 and S % K == 0
    NSTAGE = EPW // S
    NCHUNK_S = S // K
    assert NCHUNK_S >= 4 and (NCHUNK_S - 4) % 3 == 0
    NGRP = (NCHUNK_S - 4) // 3
    KV = K // L

    @functools.partial(
        pl.kernel,
        out_type=jax.ShapeDtypeStruct((NC, n_nodes, hidden), jnp.float32),
        mesh=_sc_mesh(),
        scratch_types=[
            pltpu.VMEM_SHARED((n_nodes, hidden), jnp.float32),
            pltpu.VMEM((K, hidden), jnp.float32),
            pltpu.VMEM((K, hidden), jnp.float32),
            pltpu.VMEM((K, hidden), jnp.float32),
            pltpu.VMEM((K,), jnp.int32),
            pltpu.VMEM((K,), jnp.int32),
            pltpu.VMEM((K,), jnp.int32),
            pltpu.VMEM((K,), jnp.int32),
            pltpu.VMEM((K,), jnp.int32),
            pltpu.VMEM((K,), jnp.int32),
            pltpu.VMEM((2 * S,), jnp.int32),
            pltpu.VMEM((2 * S,), jnp.int32),
            pltpu.VMEM((2 * S,), jnp.float32),
            pltpu.SemaphoreType.DMA,
            pltpu.SemaphoreType.DMA,
            pltpu.SemaphoreType.DMA,
            pltpu.SemaphoreType.DMA,
            pltpu.SemaphoreType.DMA,
            pltpu.SemaphoreType.DMA,
            pltpu.SemaphoreType.DMA,
        ],
        compiler_params=pltpu.CompilerParams(needs_layout_passes=False),
    )
    def agg_kernel(tbl_hbm, gidx_hbm, dst_hbm, scale_hbm, zeros_hbm,
                   out_hbm, acc_sh, rows0, rows1, rows2, gv0, gv1, gv2,
                   dv0, dv1, dv2, gidx_all, dst_all, scale_all,
                   sg0, sg1, sg2, ss0, ss1, ss2, sstage):
        cc = lax.axis_index("c")
        ss = lax.axis_index("s")
        wid = cc * NS + ss
        base = wid * EPW
        ROWS = (rows0, rows1, rows2)
        GV = (gv0, gv1, gv2)
        DV = (dv0, dv1, dv2)
        SG = (sg0, sg1, sg2)
        SS = (ss0, ss1, ss2)

        def zero_chunk(k, carry):
            cid = ss + k * NS

            @pl.when(cid < NZCH)
            def _():
                pltpu.sync_copy(zeros_hbm.at[pl.ds(cid * ZR, ZR)],
                                acc_sh.at[pl.ds(cid * ZR, ZR)])
            return carry

        lax.fori_loop(0, NZROUND, zero_chunk, 0)
        plsc.subcore_barrier()

        def fill(buf, src, off):
            for v in range(KV):
                buf[pl.ds(v * L, L)] = src[pl.ds(off + v * L, L)]

        def fire_gather(b, i, sb):
            fill(GV[b], gidx_all, sb + i * K)
            pltpu.async_copy(tbl_hbm.at[GV[b]], ROWS[b], SG[b])

        def wait_gather(b):
            pltpu.make_async_copy(tbl_hbm.at[GV[b]], ROWS[b], SG[b]).wait()

        def scale_scatter(b, i, sb):
            rows = ROWS[b]

            def scale_grp(jg, c2):
                sc16 = scale_all[pl.ds(sb + i * K + jg * L, L)]
                for p in range(L):
                    sc = jnp.take_along_axis(
                        sc16, jnp.full((L,), p, jnp.int32), 0)
                    j = jg * L + p
                    for h in range(HV):
                        rows[j, pl.ds(h * L, L)] = (
                            rows[j, pl.ds(h * L, L)] * sc)
                return c2

            lax.fori_loop(0, K // L, scale_grp, 0, unroll=True)
            fill(DV[b], dst_all, sb + i * K)
            pltpu.async_copy(rows, acc_sh.at[DV[b]], SS[b], add=True)

        def wait_scatter(b):
            pltpu.make_async_copy(ROWS[b], acc_sh.at[DV[b]], SS[b]).wait()

        def stage_dma_descs(st, half):
            soff = base + st * S
            hoff = half * S
            return (
                pltpu.make_async_copy(gidx_hbm.at[pl.ds(soff, S)],
                                      gidx_all.at[pl.ds(hoff, S)], sstage),
                pltpu.make_async_copy(dst_hbm.at[pl.ds(soff, S)],
                                      dst_all.at[pl.ds(hoff, S)], sstage),
                pltpu.make_async_copy(scale_hbm.at[pl.ds(soff, S)],
                                      scale_all.at[pl.ds(hoff, S)], sstage),
            )

        pltpu.sync_copy(gidx_hbm.at[pl.ds(base, S)],
                        gidx_all.at[pl.ds(0, S)])
        pltpu.sync_copy(dst_hbm.at[pl.ds(base, S)],
                        dst_all.at[pl.ds(0, S)])
        pltpu.sync_copy(scale_hbm.at[pl.ds(base, S)],
                        scale_all.at[pl.ds(0, S)])

        def stage_round(st, carry):
            sb = (st % 2) * S

            @pl.when(st > 0)
            def _():
                for d in stage_dma_descs(st, st % 2):
                    d.wait()

            fire_gather(0, 0, sb)
            fire_gather(1, 1, sb)
            fire_gather(2, 2, sb)

            @pl.when(st < NSTAGE - 1)
            def _():
                for d in stage_dma_descs(st + 1, (st + 1) % 2):
                    d.start()

            wait_gather(0)
            scale_scatter(0, 0, sb)
            wait_gather(1)
            scale_scatter(1, 1, sb)
            wait_scatter(0)
            fire_gather(0, 3, sb)

            def grp(g, c2):
                i0 = 2 + g * 3
                for u in range(3):
                    w = (2 + u) % 3
                    nb = (4 + u) % 3
                    wait_gather(w)
                    scale_scatter(w, i0 + u, sb)
                    wait_scatter(nb)
                    fire_gather(nb, i0 + u + 2, sb)
                return c2

            lax.fori_loop(0, NGRP, grp, 0)
            wait_gather((NCHUNK_S - 2) % 3)
            scale_scatter((NCHUNK_S - 2) % 3, NCHUNK_S - 2, sb)
            wait_gather((NCHUNK_S - 1) % 3)
            scale_scatter((NCHUNK_S - 1) % 3, NCHUNK_S - 1, sb)
            wait_scatter(0)
            wait_scatter(1)
            wait_scatter(2)
            return carry

        lax.fori_loop(0, NSTAGE, stage_round, 0)
        plsc.subcore_barrier()

        def out_chunk(k, carry):
            cid = ss + k * NS

            @pl.when(cid < NZCH)
            def _():
                pltpu.sync_copy(acc_sh.at[pl.ds(cid * ZR, ZR)],
                                out_hbm.at[cc, pl.ds(cid * ZR, ZR)])
            return carry

        lax.fori_loop(0, NZROUND, out_chunk, 0)

    return agg_kernel


def _mm_body(a_ref, b_ref, c_ref, bias_ref, w_ref, h_ref, x_ref):
    v = a_ref[...] + b_ref[...] + c_ref[...] + bias_ref[...][None, :]
    x = jnp.where(v > 0, v, jnp.exp(v) - 1.0)
    h_ref[0] = jnp.dot(x, w_ref[0], preferred_element_type=jnp.float32)

    @pl.when(pl.program_id(1) == 0)
    def _():
        x_ref[...] = x


def _rel_matmuls(a, b, c, bias, w_all):
    n, h = a.shape
    nr = w_all.shape[0]
    blk = 1000
    assert n % blk == 0
    row_spec = pl.BlockSpec((blk, h), lambda i, r: (i, 0))
    return pl.pallas_call(
        _mm_body,
        grid=(n // blk, nr),
        in_specs=[
            row_spec,
            row_spec,
            row_spec,
            pl.BlockSpec((h,), lambda i, r: (0,)),
            pl.BlockSpec((1, h, h), lambda i, r: (r, 0, 0)),
        ],
        out_specs=[
            pl.BlockSpec((1, blk, h), lambda i, r: (r, i, 0)),
            row_spec,
        ],
        out_shape=[
            jax.ShapeDtypeStruct((nr, n, h), jnp.float32),
            jax.ShapeDtypeStruct((n, h), jnp.float32),
        ],
    )(a, b, c, bias, w_all)


def _fuse_mm_body(a_ref, b_ref, x_ref, w_ref, bias_ref, o_ref):
    v = (a_ref[...] + b_ref[...] + bias_ref[...][None, :]
         + jnp.dot(x_ref[...], w_ref[...],
                   preferred_element_type=jnp.float32))
    o_ref[...] = jnp.where(v > 0, v, jnp.exp(v) - 1.0)


def _fused_elu_mm(a, b, x, w, bias):
    n, h = a.shape
    blk = 1000
    assert n % blk == 0
    return pl.pallas_call(
        _fuse_mm_body,
        grid=(n // blk,),
        in_specs=[
            pl.BlockSpec((blk, h), lambda i: (i, 0)),
            pl.BlockSpec((blk, h), lambda i: (i, 0)),
            pl.BlockSpec((blk, h), lambda i: (i, 0)),
            pl.BlockSpec((h, h), lambda i: (0, 0)),
            pl.BlockSpec((h,), lambda i: (0,)),
        ],
        out_specs=pl.BlockSpec((blk, h), lambda i: (i, 0)),
        out_shape=jax.ShapeDtypeStruct((n, h), jnp.float32),
    )(a, b, x, w, bias)


def kernel(weight1, root1, bias1, weight2, root2, bias2, edge_index,
           edge_type):
    n_rel, n_nodes, hidden = weight1.shape
    n_edges = edge_type.shape[0]
    src = edge_index[0]
    dst = edge_index[1]

    cidx = edge_type * n_nodes + dst
    gidx = edge_type * n_nodes + src

    n_seg = n_rel * n_nodes
    nrow_pad = pl.cdiv(pl.cdiv(n_seg, 128), 128) * 128
    zeros_seg = jnp.zeros((nrow_pad, 128), jnp.float32)
    zeros_acc = jnp.zeros((n_nodes, hidden), jnp.float32)

    scale = _make_scale_kernel(n_edges, n_seg)(cidx, zeros_seg)

    tbl1 = weight1.reshape(n_seg, hidden)
    p1 = _make_agg_kernel(n_seg, n_nodes, hidden, n_edges)(
        tbl1, gidx, dst, scale, zeros_acc)

    h_all, x = _rel_matmuls(p1[0], p1[1], root1, bias1, weight2)
    tbl2 = h_all.reshape(n_seg, hidden)
    p2 = _make_agg_kernel(n_seg, n_nodes, hidden, n_edges)(
        tbl2, gidx, dst, scale, zeros_acc)
    x2 = _fused_elu_mm(p2[0], p2[1], x, root2, bias2)
    return x2


# speedup vs baseline: 1.3785x; 1.3785x over previous
import functools

import jax
import jax.numpy as jnp
from jax import lax
from jax.experimental import pallas as pl
from jax.experimental.pallas import tpu as pltpu
from jax.experimental.pallas import tpu_sc as plsc

NC = 2
NS = 16
NW = NC * NS
L = 16


def _sc_mesh():
    return plsc.VectorSubcoreMesh(
        core_axis_name="c", subcore_axis_name="s", num_cores=NC,
        num_subcores=NS)


@functools.lru_cache(maxsize=None)
def _make_scale_kernel(n_edges: int, n_seg: int):
    EPW = n_edges // NW
    CPW = n_edges // NS
    assert EPW % L == 0 and n_seg % L == 0
    RW = 128
    NROW = pl.cdiv(n_seg, RW)
    NROW_PAD = pl.cdiv(NROW, RW) * RW
    SEG_PAD = NROW_PAD * RW
    RPT = NROW_PAD // NS

    @functools.partial(
        pl.kernel,
        out_type=jax.ShapeDtypeStruct((n_edges,), jnp.float32),
        mesh=_sc_mesh(),
        scratch_types=[
            pltpu.VMEM_SHARED((NROW_PAD, RW), jnp.float32),
            pltpu.VMEM((NROW_PAD, RW), jnp.float32),
            pltpu.VMEM((CPW,), jnp.int32),
            pltpu.VMEM((EPW,), jnp.float32),
            pltpu.VMEM((RW,), jnp.int32),
            pltpu.VMEM((RPT, RW), jnp.float32),
        ],
        compiler_params=pltpu.CompilerParams(needs_layout_passes=False),
    )
    def scale_kernel(cidx_hbm, zeros_hbm, scale_hbm, shared_sh, counts_v,
                     stage_v, sbuf_v, ridx_v, work_v):
        cc = lax.axis_index("c")
        ss = lax.axis_index("s")
        wid = cc * NS + ss
        base = wid * EPW

        pltpu.sync_copy(zeros_hbm, counts_v)
        pltpu.sync_copy(zeros_hbm.at[pl.ds(ss * RPT, RPT)], work_v)
        pltpu.sync_copy(work_v, shared_sh.at[pl.ds(ss * RPT, RPT)])

        pltpu.sync_copy(cidx_hbm.at[pl.ds(ss * CPW, CPW)], stage_v)
        ones = jnp.ones((L,), jnp.float32)
        UNR = 5
        assert CPW % (L * UNR) == 0

        def count_vec(j, c2):
            for u in range(UNR):
                idx = stage_v[pl.ds((j * UNR + u) * L, L)]
                plsc.addupdate_scatter(
                    counts_v, [lax.shift_right_logical(idx, 7),
                               lax.bitwise_and(idx, RW - 1)], ones)
            return c2

        lax.fori_loop(0, CPW // (L * UNR), count_vec, 0)
        plsc.subcore_barrier()

        def reduce_chunk(r, carry):
            rbase = r * RW
            for v in range(RW // L):
                ridx_v[pl.ds(v * L, L)] = (
                    lax.iota(jnp.int32, L) + rbase + v * L)
            pltpu.sync_copy(counts_v.at[pl.ds(rbase, RW)],
                            shared_sh.at[ridx_v], add=True)
            return carry

        lax.fori_loop(0, NROW_PAD // RW, reduce_chunk, 0)
        plsc.subcore_barrier()

        pltpu.sync_copy(shared_sh.at[pl.ds(ss * RPT, RPT)], work_v)

        def inv_row(r, carry):
            for v in range(RW // L):
                val = work_v[r, pl.ds(v * L, L)]
                work_v[r, pl.ds(v * L, L)] = 1.0 / jnp.maximum(val, 1.0)
            return carry

        lax.fori_loop(0, RPT, inv_row, 0)
        pltpu.sync_copy(work_v, shared_sh.at[pl.ds(ss * RPT, RPT)])
        plsc.subcore_barrier()

        pltpu.sync_copy(shared_sh, counts_v)
        pltpu.sync_copy(cidx_hbm.at[pl.ds(base, EPW)],
                        stage_v.at[pl.ds(0, EPW)])

        def scale_vec(i, carry):
            idx = stage_v[pl.ds(i * L, L)]
            sbuf_v[pl.ds(i * L, L)] = plsc.load_gather(
                counts_v, [lax.shift_right_logical(idx, 7),
                           lax.bitwise_and(idx, RW - 1)])
            return carry

        lax.fori_loop(0, EPW // L, scale_vec, 0)
        pltpu.sync_copy(sbuf_v, scale_hbm.at[pl.ds(base, EPW)])

    return scale_kernel


@functools.lru_cache(maxsize=None)
def _make_agg_kernel(tbl_rows: int, n_nodes: int, hidden: int,
                     n_edges: int):
    K = 80
    EPW = n_edges // NW
    assert EPW % K == 0 and K % 8 == 0
    NCHUNK = EPW // K
    ZR = 80
    NZCH = pl.cdiv(n_nodes, ZR)
    NZROUND = pl.cdiv(NZCH, NS)
    assert n_nodes % ZR == 0
    HV = hidden // L

    S = 2000
    assert EPW % S == 0 and S % K == 0
    NSTAGE = EPW // S
    NCHUNK_S = S // K
    assert NCHUNK_S >= 4 and (NCHUNK_S - 4) % 3 == 0
    NGRP = (NCHUNK_S - 4) // 3
    KV = K // L

    @functools.partial(
        pl.kernel,
        out_type=jax.ShapeDtypeStruct((NC, n_nodes, hidden), jnp.float32),
        mesh=_sc_mesh(),
        scratch_types=[
            pltpu.VMEM_SHARED((n_nodes, hidden), jnp.float32),
            pltpu.VMEM((K, hidden), jnp.float32),
            pltpu.VMEM((K, hidden), jnp.float32),
            pltpu.VMEM((K, hidden), jnp.float32),
            pltpu.VMEM((K,), jnp.int32),
            pltpu.VMEM((K,), jnp.int32),
            pltpu.VMEM((K,), jnp.int32),
            pltpu.VMEM((K,), jnp.int32),
            pltpu.VMEM((K,), jnp.int32),
            pltpu.VMEM((K,), jnp.int32),
            pltpu.VMEM((2 * S,), jnp.int32),
            pltpu.VMEM((2 * S,), jnp.int32),
            pltpu.VMEM((2 * S,), jnp.float32),
            pltpu.SemaphoreType.DMA,
            pltpu.SemaphoreType.DMA,
            pltpu.SemaphoreType.DMA,
            pltpu.SemaphoreType.DMA,
            pltpu.SemaphoreType.DMA,
            pltpu.SemaphoreType.DMA,
            pltpu.SemaphoreType.DMA,
        ],
        compiler_params=pltpu.CompilerParams(needs_layout_passes=False),
    )
    def agg_kernel(tbl_hbm, gidx_hbm, dst_hbm, scale_hbm, zeros_hbm,
                   out_hbm, acc_sh, rows0, rows1, rows2, gv0, gv1, gv2,
                   dv0, dv1, dv2, gidx_all, dst_all, scale_all,
                   sg0, sg1, sg2, ss0, ss1, ss2, sstage):
        cc = lax.axis_index("c")
        ss = lax.axis_index("s")
        wid = cc * NS + ss
        base = wid * EPW
        ROWS = (rows0, rows1, rows2)
        GV = (gv0, gv1, gv2)
        DV = (dv0, dv1, dv2)
        SG = (sg0, sg1, sg2)
        SS = (ss0, ss1, ss2)

        def zero_chunk(k, carry):
            cid = ss + k * NS

            @pl.when(cid < NZCH)
            def _():
                pltpu.sync_copy(zeros_hbm.at[pl.ds(cid * ZR, ZR)],
                                acc_sh.at[pl.ds(cid * ZR, ZR)])
            return carry

        lax.fori_loop(0, NZROUND, zero_chunk, 0)
        plsc.subcore_barrier()

        def fill(buf, src, off):
            for v in range(KV):
                buf[pl.ds(v * L, L)] = src[pl.ds(off + v * L, L)]

        def fire_gather(b, i, sb):
            fill(GV[b], gidx_all, sb + i * K)
            pltpu.async_copy(tbl_hbm.at[GV[b]], ROWS[b], SG[b])

        def wait_gather(b):
            pltpu.make_async_copy(tbl_hbm.at[GV[b]], ROWS[b], SG[b]).wait()

        def scale_scatter(b, i, sb):
            rows = ROWS[b]

            def scale_grp(jg, c2):
                sc16 = scale_all[pl.ds(sb + i * K + jg * L, L)]
                for p in range(L):
                    sc = jnp.take_along_axis(
                        sc16, jnp.full((L,), p, jnp.int32), 0)
                    j = jg * L + p
                    for h in range(HV):
                        rows[j, pl.ds(h * L, L)] = (
                            rows[j, pl.ds(h * L, L)] * sc)
                return c2

            lax.fori_loop(0, K // L, scale_grp, 0)
            fill(DV[b], dst_all, sb + i * K)
            pltpu.async_copy(rows, acc_sh.at[DV[b]], SS[b], add=True)

        def wait_scatter(b):
            pltpu.make_async_copy(ROWS[b], acc_sh.at[DV[b]], SS[b]).wait()

        def stage_dma_descs(st, half):
            soff = base + st * S
            hoff = half * S
            return (
                pltpu.make_async_copy(gidx_hbm.at[pl.ds(soff, S)],
                                      gidx_all.at[pl.ds(hoff, S)], sstage),
                pltpu.make_async_copy(dst_hbm.at[pl.ds(soff, S)],
                                      dst_all.at[pl.ds(hoff, S)], sstage),
                pltpu.make_async_copy(scale_hbm.at[pl.ds(soff, S)],
                                      scale_all.at[pl.ds(hoff, S)], sstage),
            )

        pltpu.sync_copy(gidx_hbm.at[pl.ds(base, S)],
                        gidx_all.at[pl.ds(0, S)])
        pltpu.sync_copy(dst_hbm.at[pl.ds(base, S)],
                        dst_all.at[pl.ds(0, S)])
        pltpu.sync_copy(scale_hbm.at[pl.ds(base, S)],
                        scale_all.at[pl.ds(0, S)])

        def stage_round(st, carry):
            sb = (st % 2) * S

            @pl.when(st > 0)
            def _():
                for d in stage_dma_descs(st, st % 2):
                    d.wait()

            fire_gather(0, 0, sb)
            fire_gather(1, 1, sb)
            fire_gather(2, 2, sb)

            @pl.when(st < NSTAGE - 1)
            def _():
                for d in stage_dma_descs(st + 1, (st + 1) % 2):
                    d.start()

            wait_gather(0)
            scale_scatter(0, 0, sb)
            wait_gather(1)
            scale_scatter(1, 1, sb)
            wait_scatter(0)
            fire_gather(0, 3, sb)

            def grp(g, c2):
                i0 = 2 + g * 3
                for u in range(3):
                    w = (2 + u) % 3
                    nb = (4 + u) % 3
                    wait_gather(w)
                    scale_scatter(w, i0 + u, sb)
                    wait_scatter(nb)
                    fire_gather(nb, i0 + u + 2, sb)
                return c2

            lax.fori_loop(0, NGRP, grp, 0)
            wait_gather((NCHUNK_S - 2) % 3)
            scale_scatter((NCHUNK_S - 2) % 3, NCHUNK_S - 2, sb)
            wait_gather((NCHUNK_S - 1) % 3)
            scale_scatter((NCHUNK_S - 1) % 3, NCHUNK_S - 1, sb)
            wait_scatter(0)
            wait_scatter(1)
            wait_scatter(2)
            return carry

        lax.fori_loop(0, NSTAGE, stage_round, 0)
        plsc.subcore_barrier()

        def out_chunk(k, carry):
            cid = ss + k * NS

            @pl.when(cid < NZCH)
            def _():
                pltpu.sync_copy(acc_sh.at[pl.ds(cid * ZR, ZR)],
                                out_hbm.at[cc, pl.ds(cid * ZR, ZR)])
            return carry

        lax.fori_loop(0, NZROUND, out_chunk, 0)

    return agg_kernel


def _mm_body(a_ref, b_ref, c_ref, bias_ref, w_ref, h_ref, x_ref):
    v = a_ref[...] + b_ref[...] + c_ref[...] + bias_ref[...][None, :]
    x = jnp.where(v > 0, v, jnp.exp(v) - 1.0)
    h_ref[0] = jnp.dot(x, w_ref[0], preferred_element_type=jnp.float32)

    @pl.when(pl.program_id(1) == 0)
    def _():
        x_ref[...] = x


def _rel_matmuls(a, b, c, bias, w_all):
    n, h = a.shape
    nr = w_all.shape[0]
    blk = 1000
    assert n % blk == 0
    row_spec = pl.BlockSpec((blk, h), lambda i, r: (i, 0))
    return pl.pallas_call(
        _mm_body,
        grid=(n // blk, nr),
        in_specs=[
            row_spec,
            row_spec,
            row_spec,
            pl.BlockSpec((h,), lambda i, r: (0,)),
            pl.BlockSpec((1, h, h), lambda i, r: (r, 0, 0)),
        ],
        out_specs=[
            pl.BlockSpec((1, blk, h), lambda i, r: (r, i, 0)),
            row_spec,
        ],
        out_shape=[
            jax.ShapeDtypeStruct((nr, n, h), jnp.float32),
            jax.ShapeDtypeStruct((n, h), jnp.float32),
        ],
    )(a, b, c, bias, w_all)


def _fuse_mm_body(a_ref, b_ref, x_ref, w_ref, bias_ref, o_ref):
    v = (a_ref[...] + b_ref[...] + bias_ref[...][None, :]
         + jnp.dot(x_ref[...], w_ref[...],
                   preferred_element_type=jnp.float32))
    o_ref[...] = jnp.where(v > 0, v, jnp.exp(v) - 1.0)


def _fused_elu_mm(a, b, x, w, bias):
    n, h = a.shape
    blk = 1000
    assert n % blk == 0
    return pl.pallas_call(
        _fuse_mm_body,
        grid=(n // blk,),
        in_specs=[
            pl.BlockSpec((blk, h), lambda i: (i, 0)),
            pl.BlockSpec((blk, h), lambda i: (i, 0)),
            pl.BlockSpec((blk, h), lambda i: (i, 0)),
            pl.BlockSpec((h, h), lambda i: (0, 0)),
            pl.BlockSpec((h,), lambda i: (0,)),
        ],
        out_specs=pl.BlockSpec((blk, h), lambda i: (i, 0)),
        out_shape=jax.ShapeDtypeStruct((n, h), jnp.float32),
    )(a, b, x, w, bias)


def kernel(weight1, root1, bias1, weight2, root2, bias2, edge_index,
           edge_type):
    n_rel, n_nodes, hidden = weight1.shape
    n_edges = edge_type.shape[0]
    src = edge_index[0]
    dst = edge_index[1]

    cidx = edge_type * n_nodes + dst
    gidx = edge_type * n_nodes + src

    n_seg = n_rel * n_nodes
    nrow_pad = pl.cdiv(pl.cdiv(n_seg, 128), 128) * 128
    zeros_seg = jnp.zeros((nrow_pad, 128), jnp.float32)
    zeros_acc = jnp.zeros((n_nodes, hidden), jnp.float32)

    scale = _make_scale_kernel(n_edges, n_seg)(cidx, zeros_seg)

    tbl1 = weight1.reshape(n_seg, hidden)
    p1 = _make_agg_kernel(n_seg, n_nodes, hidden, n_edges)(
        tbl1, gidx, dst, scale, zeros_acc)

    h_all, x = _rel_matmuls(p1[0], p1[1], root1, bias1, weight2)
    tbl2 = h_all.reshape(n_seg, hidden)
    p2 = _make_agg_kernel(n_seg, n_nodes, hidden, n_edges)(
        tbl2, gidx, dst, scale, zeros_acc)
    x2 = _fused_elu_mm(p2[0], p2[1], x, root2, bias2)
    return x2
